# baseline (speedup 1.0000x reference)
import jax
import jax.numpy as jnp
from jax.experimental import pallas as pl
from jax.experimental.pallas import tpu as pltpu

_IN = 144
_CELLS = 36
_NC = 4
_SLAB_COLS = 17
_BT = 4096


def _mlp_kernel(nm_ref, x_ref, w1t_ref, slab_ref, o_ref):
    w1 = w1t_ref[...]
    slab = slab_ref[...]

    h = None
    for m in range(_CELLS):
        w_m = jax.lax.slice(w1, (4 * m, 0), (4 * m + 4, _NC))
        x_m = x_ref[m]
        part = jax.lax.dot_general(w_m, x_m, (((0,), (0,)), ((), ())),
                                   preferred_element_type=jnp.float32)
        h = part if h is None else h + part
    h = jax.nn.sigmoid(h + slab[:, 0:1])

    def dense(hh, w, b):
        out = b + w[:, 0:1] * hh[0:1, :]
        out = out + w[:, 1:2] * hh[1:2, :]
        out = out + w[:, 2:3] * hh[2:3, :]
        out = out + w[:, 3:4] * hh[3:4, :]
        return out

    h = jax.nn.sigmoid(dense(h, slab[:, 1:5], slab[:, 5:6]))
    h = jax.nn.sigmoid(dense(h, slab[:, 6:10], slab[:, 10:11]))
    logits = dense(h, slab[:, 11:15], slab[:, 15:16])

    m = jnp.max(logits, axis=0, keepdims=True)
    e = jnp.exp(logits - m)
    probs = e / jnp.sum(e, axis=0, keepdims=True)

    nm = nm_ref[0]
    mask = (jax.lax.broadcasted_iota(jnp.int32, (_NC, 1), 0) < nm
            ).astype(jnp.float32)
    o_ref[...] = probs * mask


def _forward(xt, w1t, slab, num_moves, b):
    nm = jnp.reshape(num_moves, (1,)).astype(jnp.int32)
    bt = _BT if b % _BT == 0 else (128 if b % 128 == 0 else b)
    out_t = pl.pallas_call(
        _mlp_kernel,
        out_shape=jax.ShapeDtypeStruct((_NC, b), jnp.float32),
        grid_spec=pltpu.PrefetchScalarGridSpec(
            num_scalar_prefetch=1,
            grid=(b // bt,),
            in_specs=[
                pl.BlockSpec((_CELLS, _NC, bt), lambda i, n: (0, 0, i)),
                pl.BlockSpec((_IN, _NC), lambda i, n: (0, 0)),
                pl.BlockSpec((_NC, _SLAB_COLS), lambda i, n: (0, 0)),
            ],
            out_specs=pl.BlockSpec((_NC, bt), lambda i, n: (0, i)),
        ),
        compiler_params=pltpu.CompilerParams(
            dimension_semantics=("parallel",),
            vmem_limit_bytes=56 * 1024 * 1024),
    )(nm, xt, w1t, slab)
    return out_t


def kernel(x, w1t, slab, num_moves):
    x = jnp.asarray(x, jnp.float32)
    single = x.ndim in (1, 3)
    if single:
        x = x.reshape(1, 6, 6, _NC)
    b = x.shape[0]
    xt = x.transpose(1, 2, 3, 0).reshape(_CELLS, _NC, b)
    out_t = _forward(xt, w1t, slab, num_moves, b)
    out = out_t.T
    return out[0] if single else out


# speedup vs baseline: 1.1941x; 1.1941x over previous
import jax
import jax.numpy as jnp
from jax.experimental import pallas as pl
from jax.experimental.pallas import tpu as pltpu

_IN = 144
_CELLS = 36
_NC = 4
_SLAB_COLS = 17
_BT = 16384


def _mlp_kernel(nm_ref, x_ref, w1t_ref, slab_ref, o_ref):
    w1 = w1t_ref[...]
    slab = slab_ref[...]

    h = None
    for m in range(_CELLS):
        w_m = jax.lax.slice(w1, (4 * m, 0), (4 * m + 4, _NC))
        x_m = x_ref[m]
        part = jax.lax.dot_general(w_m, x_m, (((0,), (0,)), ((), ())),
                                   preferred_element_type=jnp.float32)
        h = part if h is None else h + part
    h = jax.nn.sigmoid(h + slab[:, 0:1])

    def dense(hh, w, b):
        out = b + w[:, 0:1] * hh[0:1, :]
        out = out + w[:, 1:2] * hh[1:2, :]
        out = out + w[:, 2:3] * hh[2:3, :]
        out = out + w[:, 3:4] * hh[3:4, :]
        return out

    h = jax.nn.sigmoid(dense(h, slab[:, 1:5], slab[:, 5:6]))
    h = jax.nn.sigmoid(dense(h, slab[:, 6:10], slab[:, 10:11]))
    logits = dense(h, slab[:, 11:15], slab[:, 15:16])

    m = jnp.max(logits, axis=0, keepdims=True)
    e = jnp.exp(logits - m)
    probs = e / jnp.sum(e, axis=0, keepdims=True)

    nm = nm_ref[0]
    mask = (jax.lax.broadcasted_iota(jnp.int32, (_NC, 1), 0) < nm
            ).astype(jnp.float32)
    o_ref[...] = probs * mask


def _forward(xt, w1t, slab, num_moves, b):
    nm = jnp.reshape(num_moves, (1,)).astype(jnp.int32)
    bt = _BT if b % _BT == 0 else (128 if b % 128 == 0 else b)
    out_t = pl.pallas_call(
        _mlp_kernel,
        out_shape=jax.ShapeDtypeStruct((_NC, b), jnp.float32),
        grid_spec=pltpu.PrefetchScalarGridSpec(
            num_scalar_prefetch=1,
            grid=(b // bt,),
            in_specs=[
                pl.BlockSpec((_CELLS, _NC, bt), lambda i, n: (0, 0, i)),
                pl.BlockSpec((_IN, _NC), lambda i, n: (0, 0)),
                pl.BlockSpec((_NC, _SLAB_COLS), lambda i, n: (0, 0)),
            ],
            out_specs=pl.BlockSpec((_NC, bt), lambda i, n: (0, i)),
        ),
        compiler_params=pltpu.CompilerParams(
            dimension_semantics=("parallel",),
            vmem_limit_bytes=56 * 1024 * 1024),
    )(nm, xt, w1t, slab)
    return out_t


def kernel(x, w1t, slab, num_moves):
    x = jnp.asarray(x, jnp.float32)
    single = x.ndim in (1, 3)
    if single:
        x = x.reshape(1, 6, 6, _NC)
    b = x.shape[0]
    xt = x.transpose(1, 2, 3, 0).reshape(_CELLS, _NC, b)
    out_t = _forward(xt, w1t, slab, num_moves, b)
    out = out_t.T
    return out[0] if single else out
